# baseline (speedup 1.0000x reference)
import functools

import jax
import jax.numpy as jnp
from jax import lax
from jax.experimental import pallas as pl
from jax.experimental.pallas import tpu as pltpu
from jax.experimental.pallas import tpu_sc as plsc

N = 8192
K = 1024
D = 32
BN = 1024
NB = N // BN

NC = 2
NS = 16
NW = NC * NS
BPW = N // NW
CHUNK = 128
NCHUNK = BPW // CHUNK


def _tc_body(z_ref, emb_ref, z2_ref, e2_ref, idx_ref, loss_ref):
    i = pl.program_id(0)
    z = z_ref[...]
    e = emb_ref[...]
    zv = lax.dot_general(z, e, (((1,), (1,)), ((), ())),
                         preferred_element_type=jnp.float32)
    d = (z2_ref[...] + e2_ref[...]) - 2.0 * zv
    md = jnp.min(d, axis=1, keepdims=True)
    lane = lax.broadcasted_iota(jnp.int32, d.shape, 1).astype(jnp.float32)
    hit = jnp.where(d == md, lane, jnp.float32(K))
    idx = jnp.min(hit, axis=1, keepdims=True).astype(jnp.int32)
    idx_ref[...] = idx.reshape(BN // CHUNK, CHUNK)

    @pl.when(i == 0)
    def _():
        loss_ref[0, 0] = 0.0

    loss_ref[0, 0] = loss_ref[0, 0] + jnp.sum(md)

    @pl.when(i == NB - 1)
    def _():
        loss_ref[0, 0] = loss_ref[0, 0] * (1.25 / (N * D))


_tc_call = pl.pallas_call(
    _tc_body,
    grid=(NB,),
    compiler_params=pltpu.CompilerParams(
        allow_input_fusion=[True, True, True, True]),
    in_specs=[
        pl.BlockSpec((BN, D), lambda i: (i, 0)),
        pl.BlockSpec((K, D), lambda i: (0, 0)),
        pl.BlockSpec((BN, 1), lambda i: (i, 0)),
        pl.BlockSpec((1, K), lambda i: (0, 0)),
    ],
    out_specs=[
        pl.BlockSpec((BN // CHUNK, CHUNK), lambda i: (i, 0)),
        pl.BlockSpec((1, 1), lambda i: (0, 0), memory_space=pltpu.SMEM),
    ],
    out_shape=[
        jax.ShapeDtypeStruct((N // CHUNK, CHUNK), jnp.int32),
        jax.ShapeDtypeStruct((1, 1), jnp.float32),
    ],
)


def _sc_body(emb_hbm, idx_hbm, out_hbm, idx_v, rows_v, sem):
    wid = lax.axis_index("s") * NC + lax.axis_index("c")
    pltpu.sync_copy(idx_hbm.at[pl.ds(wid * NCHUNK, NCHUNK)], idx_v)
    cps = [
        pltpu.async_copy(emb_hbm.at[idx_v.at[c]],
                         rows_v.at[pl.ds(c * CHUNK, CHUNK)], sem)
        for c in range(NCHUNK)
    ]
    for cp in cps:
        cp.wait()
    pltpu.sync_copy(rows_v, out_hbm.at[pl.ds(wid * BPW, BPW)])


@functools.cache
def _sc_gather():
    return pl.kernel(
        _sc_body,
        mesh=plsc.VectorSubcoreMesh(core_axis_name="c", subcore_axis_name="s"),
        out_type=jax.ShapeDtypeStruct((N, D), jnp.float32),
        compiler_params=pltpu.CompilerParams(use_tc_tiling_on_sc=False),
        scratch_types=[
            pltpu.VMEM((NCHUNK, CHUNK), jnp.int32),
            pltpu.VMEM((BPW, D), jnp.float32),
            pltpu.SemaphoreType.DMA,
        ],
    )


def kernel(z, emb):
    z2 = jnp.sum(z * z, axis=1, keepdims=True)
    e2 = jnp.sum(emb * emb, axis=1)[None, :]
    idx2d, loss = _tc_call(z, emb, z2, e2)
    nearest_idx = idx2d.reshape(N)
    z_q = _sc_gather()(emb, idx2d)
    return (z_q, nearest_idx, loss[0, 0])


# speedup vs baseline: 1.1489x; 1.1489x over previous
import functools

import jax
import jax.numpy as jnp
from jax import lax
from jax.experimental import pallas as pl
from jax.experimental.pallas import tpu as pltpu
from jax.experimental.pallas import tpu_sc as plsc

N = 8192
K = 1024
D = 32
BN = 1024
NB = N // BN

NC = 2
NS = 16
NW = NC * NS
BPW = N // NW
CHUNK = 128
NCHUNK = BPW // CHUNK


def _tc_body(zt_ref, emb_ref, z2_ref, e2_ref, idx_ref, loss_ref):
    i = pl.program_id(0)
    zt = zt_ref[...]
    e = emb_ref[...]
    zvt = lax.dot_general(e, zt, (((1,), (0,)), ((), ())),
                          preferred_element_type=jnp.float32)
    d = (z2_ref[...] + e2_ref[...]) - 2.0 * zvt
    md = jnp.min(d, axis=0, keepdims=True)
    code = lax.broadcasted_iota(jnp.int32, d.shape, 0).astype(jnp.float32)
    hit = jnp.where(d == md, code, jnp.float32(K))
    idx_ref[...] = jnp.min(hit, axis=0, keepdims=True).astype(jnp.int32)

    @pl.when(i == 0)
    def _():
        loss_ref[0, 0] = 0.0

    loss_ref[0, 0] = loss_ref[0, 0] + jnp.sum(md)

    @pl.when(i == NB - 1)
    def _():
        loss_ref[0, 0] = loss_ref[0, 0] * (1.25 / (N * D))


_tc_call = pl.pallas_call(
    _tc_body,
    grid=(NB,),
    compiler_params=pltpu.CompilerParams(
        allow_input_fusion=[True, True, True, True]),
    in_specs=[
        pl.BlockSpec((D, BN), lambda i: (0, i)),
        pl.BlockSpec((K, D), lambda i: (0, 0)),
        pl.BlockSpec((1, BN), lambda i: (0, i)),
        pl.BlockSpec((K, 1), lambda i: (0, 0)),
    ],
    out_specs=[
        pl.BlockSpec((1, BN), lambda i: (0, i)),
        pl.BlockSpec((1, 1), lambda i: (0, 0), memory_space=pltpu.SMEM),
    ],
    out_shape=[
        jax.ShapeDtypeStruct((1, N), jnp.int32),
        jax.ShapeDtypeStruct((1, 1), jnp.float32),
    ],
)


def _sc_body(emb_hbm, idx_hbm, out_hbm, idx_v, rows_v, sem):
    wid = lax.axis_index("s") * NC + lax.axis_index("c")
    pltpu.sync_copy(idx_hbm.at[0, pl.ds(wid * BPW, BPW)], idx_v)
    cps = [
        pltpu.async_copy(emb_hbm.at[idx_v.at[pl.ds(c * CHUNK, CHUNK)]],
                         rows_v.at[pl.ds(c * CHUNK, CHUNK)], sem)
        for c in range(NCHUNK)
    ]
    for cp in cps:
        cp.wait()
    pltpu.sync_copy(rows_v, out_hbm.at[pl.ds(wid * BPW, BPW)])


@functools.cache
def _sc_gather():
    return pl.kernel(
        _sc_body,
        mesh=plsc.VectorSubcoreMesh(core_axis_name="c", subcore_axis_name="s"),
        out_type=jax.ShapeDtypeStruct((N, D), jnp.float32),
        compiler_params=pltpu.CompilerParams(use_tc_tiling_on_sc=False),
        scratch_types=[
            pltpu.VMEM((BPW,), jnp.int32),
            pltpu.VMEM((BPW, D), jnp.float32),
            pltpu.SemaphoreType.DMA,
        ],
    )


def kernel(z, emb):
    z2 = jnp.sum(z * z, axis=1)[None, :]
    e2 = jnp.sum(emb * emb, axis=1)[:, None]
    idx2d, loss = _tc_call(z.T, emb, z2, e2)
    nearest_idx = idx2d.reshape(N)
    z_q = _sc_gather()(emb, idx2d)
    return (z_q, nearest_idx, loss[0, 0])


# speedup vs baseline: 1.1524x; 1.0030x over previous
import functools

import jax
import jax.numpy as jnp
from jax import lax
from jax.experimental import pallas as pl
from jax.experimental.pallas import tpu as pltpu
from jax.experimental.pallas import tpu_sc as plsc

N = 8192
K = 1024
D = 32
BN = 1024
NB = N // BN

NC = 2
NS = 16
NW = NC * NS
BPW = N // NW
CHUNK = 128
NCHUNK = BPW // CHUNK


def _tc_body(zt_ref, m2e_ref, z2_ref, e2_ref, idx_ref, loss_ref):
    i = pl.program_id(0)
    zt = zt_ref[...]
    m2e = m2e_ref[...]
    zvt2 = lax.dot_general(m2e, zt, (((1,), (0,)), ((), ())),
                           preferred_element_type=jnp.float32)
    d = (z2_ref[...] + e2_ref[...]) + zvt2
    md = jnp.min(d, axis=0, keepdims=True)
    code = lax.broadcasted_iota(jnp.int32, d.shape, 0).astype(jnp.float32)
    hit = jnp.where(d == md, code, jnp.float32(K))
    idx_ref[...] = jnp.min(hit, axis=0, keepdims=True).astype(jnp.int32)

    @pl.when(i == 0)
    def _():
        loss_ref[0, 0] = 0.0

    loss_ref[0, 0] = loss_ref[0, 0] + jnp.sum(md)

    @pl.when(i == NB - 1)
    def _():
        loss_ref[0, 0] = loss_ref[0, 0] * (1.25 / (N * D))


_tc_call = pl.pallas_call(
    _tc_body,
    grid=(NB,),
    compiler_params=pltpu.CompilerParams(
        allow_input_fusion=[True, True, True, True]),
    in_specs=[
        pl.BlockSpec((D, BN), lambda i: (0, i)),
        pl.BlockSpec((K, D), lambda i: (0, 0)),
        pl.BlockSpec((1, BN), lambda i: (0, i)),
        pl.BlockSpec((K, 1), lambda i: (0, 0)),
    ],
    out_specs=[
        pl.BlockSpec((1, BN), lambda i: (0, i)),
        pl.BlockSpec((1, 1), lambda i: (0, 0), memory_space=pltpu.SMEM),
    ],
    out_shape=[
        jax.ShapeDtypeStruct((1, N), jnp.int32),
        jax.ShapeDtypeStruct((1, 1), jnp.float32),
    ],
)


def _sc_body(emb_hbm, idx_hbm, out_hbm, idx_v, rows_v, sem):
    wid = lax.axis_index("s") * NC + lax.axis_index("c")
    pltpu.sync_copy(idx_hbm.at[0, pl.ds(wid * BPW, BPW)], idx_v)
    cps = [
        pltpu.async_copy(emb_hbm.at[idx_v.at[pl.ds(c * CHUNK, CHUNK)]],
                         rows_v.at[pl.ds(c * CHUNK, CHUNK)], sem)
        for c in range(NCHUNK)
    ]
    for cp in cps:
        cp.wait()
    pltpu.sync_copy(rows_v, out_hbm.at[pl.ds(wid * BPW, BPW)])


@functools.cache
def _sc_gather():
    return pl.kernel(
        _sc_body,
        mesh=plsc.VectorSubcoreMesh(core_axis_name="c", subcore_axis_name="s"),
        out_type=jax.ShapeDtypeStruct((N, D), jnp.float32),
        compiler_params=pltpu.CompilerParams(use_tc_tiling_on_sc=False),
        scratch_types=[
            pltpu.VMEM((BPW,), jnp.int32),
            pltpu.VMEM((BPW, D), jnp.float32),
            pltpu.SemaphoreType.DMA,
        ],
    )


def kernel(z, emb):
    z2 = jnp.sum(z * z, axis=1)[None, :]
    e2 = jnp.sum(emb * emb, axis=1)[:, None]
    idx2d, loss = _tc_call(z.T, -2.0 * emb, z2, e2)
    nearest_idx = idx2d.reshape(N)
    z_q = _sc_gather()(emb, idx2d)
    return (z_q, nearest_idx, loss[0, 0])


# speedup vs baseline: 1.1773x; 1.0216x over previous
import functools

import jax
import jax.numpy as jnp
from jax import lax
from jax.experimental import pallas as pl
from jax.experimental.pallas import tpu as pltpu
from jax.experimental.pallas import tpu_sc as plsc

N = 8192
K = 1024
D = 32
BN = 2048
NB = N // BN

NC = 2
NS = 16
NW = NC * NS
BPW = N // NW
CHUNK = 128
NCHUNK = BPW // CHUNK


def _tc_body(zt_ref, m2e_ref, z2_ref, e2_ref, idx_ref, loss_ref):
    i = pl.program_id(0)
    zt = zt_ref[...]
    m2e = m2e_ref[...]
    zvt2 = lax.dot_general(m2e, zt, (((1,), (0,)), ((), ())),
                           preferred_element_type=jnp.float32)
    d = (z2_ref[...] + e2_ref[...]) + zvt2
    md = jnp.min(d, axis=0, keepdims=True)
    code = lax.broadcasted_iota(jnp.int32, d.shape, 0).astype(jnp.float32)
    hit = jnp.where(d == md, code, jnp.float32(K))
    idx_ref[...] = jnp.min(hit, axis=0, keepdims=True).astype(jnp.int32)

    @pl.when(i == 0)
    def _():
        loss_ref[0, 0] = 0.0

    loss_ref[0, 0] = loss_ref[0, 0] + jnp.sum(md)

    @pl.when(i == NB - 1)
    def _():
        loss_ref[0, 0] = loss_ref[0, 0] * (1.25 / (N * D))


_tc_call = pl.pallas_call(
    _tc_body,
    grid=(NB,),
    compiler_params=pltpu.CompilerParams(
        allow_input_fusion=[True, True, True, True]),
    in_specs=[
        pl.BlockSpec((D, BN), lambda i: (0, i)),
        pl.BlockSpec((K, D), lambda i: (0, 0)),
        pl.BlockSpec((1, BN), lambda i: (0, i)),
        pl.BlockSpec((K, 1), lambda i: (0, 0)),
    ],
    out_specs=[
        pl.BlockSpec((1, BN), lambda i: (0, i)),
        pl.BlockSpec((1, 1), lambda i: (0, 0), memory_space=pltpu.SMEM),
    ],
    out_shape=[
        jax.ShapeDtypeStruct((1, N), jnp.int32),
        jax.ShapeDtypeStruct((1, 1), jnp.float32),
    ],
)


def _sc_body(emb_hbm, idx_hbm, out_hbm, idx_v, rows_v, sem):
    wid = lax.axis_index("s") * NC + lax.axis_index("c")
    pltpu.sync_copy(idx_hbm.at[0, pl.ds(wid * BPW, BPW)], idx_v)
    cps = [
        pltpu.async_copy(emb_hbm.at[idx_v.at[pl.ds(c * CHUNK, CHUNK)]],
                         rows_v.at[pl.ds(c * CHUNK, CHUNK)], sem)
        for c in range(NCHUNK)
    ]
    for cp in cps:
        cp.wait()
    pltpu.sync_copy(rows_v, out_hbm.at[pl.ds(wid * BPW, BPW)])


@functools.cache
def _sc_gather():
    return pl.kernel(
        _sc_body,
        mesh=plsc.VectorSubcoreMesh(core_axis_name="c", subcore_axis_name="s"),
        out_type=jax.ShapeDtypeStruct((N, D), jnp.float32),
        compiler_params=pltpu.CompilerParams(use_tc_tiling_on_sc=False),
        scratch_types=[
            pltpu.VMEM((BPW,), jnp.int32),
            pltpu.VMEM((BPW, D), jnp.float32),
            pltpu.SemaphoreType.DMA,
        ],
    )


def kernel(z, emb):
    z2 = jnp.sum(z * z, axis=1)[None, :]
    e2 = jnp.sum(emb * emb, axis=1)[:, None]
    idx2d, loss = _tc_call(z.T, -2.0 * emb, z2, e2)
    nearest_idx = idx2d.reshape(N)
    z_q = _sc_gather()(emb, idx2d)
    return (z_q, nearest_idx, loss[0, 0])


# speedup vs baseline: 1.1854x; 1.0069x over previous
import functools

import jax
import jax.numpy as jnp
from jax import lax
from jax.experimental import pallas as pl
from jax.experimental.pallas import tpu as pltpu
from jax.experimental.pallas import tpu_sc as plsc

N = 8192
K = 1024
D = 32
BN = 2048
NB = N // BN

NC = 2
NS = 16
NW = NC * NS
BPW = N // NW
CHUNK = 128
NCHUNK = BPW // CHUNK


def _tc_body(zt_ref, m2e_ref, z2_ref, e2_ref, idx_ref, loss_ref):
    i = pl.program_id(0)
    zt = zt_ref[...]
    m2e = m2e_ref[...]
    zvt2 = lax.dot_general(m2e, zt, (((1,), (0,)), ((), ())),
                           preferred_element_type=jnp.float32)
    e2c = e2_ref[...].reshape(K, 1)
    d = (z2_ref[...] + e2c) + zvt2
    md = jnp.min(d, axis=0, keepdims=True)
    code = lax.broadcasted_iota(jnp.int32, d.shape, 0).astype(jnp.float32)
    hit = jnp.where(d == md, code, jnp.float32(K))
    idx_ref[...] = jnp.min(hit, axis=0, keepdims=True).astype(jnp.int32)

    @pl.when(i == 0)
    def _():
        loss_ref[0, 0] = 0.0

    loss_ref[0, 0] = loss_ref[0, 0] + jnp.sum(md)

    @pl.when(i == NB - 1)
    def _():
        loss_ref[0, 0] = loss_ref[0, 0] * (1.25 / (N * D))


_tc_call = pl.pallas_call(
    _tc_body,
    grid=(NB,),
    compiler_params=pltpu.CompilerParams(
        allow_input_fusion=[True, True, True, True]),
    in_specs=[
        pl.BlockSpec((D, BN), lambda i: (0, i)),
        pl.BlockSpec((K, D), lambda i: (0, 0)),
        pl.BlockSpec((1, BN), lambda i: (0, i)),
        pl.BlockSpec((1, K), lambda i: (0, 0)),
    ],
    out_specs=[
        pl.BlockSpec((1, BN), lambda i: (0, i)),
        pl.BlockSpec((1, 1), lambda i: (0, 0), memory_space=pltpu.SMEM),
    ],
    out_shape=[
        jax.ShapeDtypeStruct((1, N), jnp.int32),
        jax.ShapeDtypeStruct((1, 1), jnp.float32),
    ],
)


def _sc_body(emb_hbm, idx_hbm, out_hbm, idx_v, rows_v, sem):
    wid = lax.axis_index("s") * NC + lax.axis_index("c")
    pltpu.sync_copy(idx_hbm.at[0, pl.ds(wid * BPW, BPW)], idx_v)
    cps = [
        pltpu.async_copy(emb_hbm.at[idx_v.at[pl.ds(c * CHUNK, CHUNK)]],
                         rows_v.at[pl.ds(c * CHUNK, CHUNK)], sem)
        for c in range(NCHUNK)
    ]
    for cp in cps:
        cp.wait()
    pltpu.sync_copy(rows_v, out_hbm.at[pl.ds(wid * BPW, BPW)])


@functools.cache
def _sc_gather():
    return pl.kernel(
        _sc_body,
        mesh=plsc.VectorSubcoreMesh(core_axis_name="c", subcore_axis_name="s"),
        out_type=jax.ShapeDtypeStruct((N, D), jnp.float32),
        compiler_params=pltpu.CompilerParams(use_tc_tiling_on_sc=False),
        scratch_types=[
            pltpu.VMEM((BPW,), jnp.int32),
            pltpu.VMEM((BPW, D), jnp.float32),
            pltpu.SemaphoreType.DMA,
        ],
    )


def kernel(z, emb):
    z2 = jnp.sum(z * z, axis=1)[None, :]
    e2 = jnp.sum(emb * emb, axis=1)[None, :]
    idx2d, loss = _tc_call(z.T, -2.0 * emb, z2, e2)
    nearest_idx = idx2d.reshape(N)
    z_q = _sc_gather()(emb, idx2d)
    return (z_q, nearest_idx, loss[0, 0])
